# baseline (speedup 1.0000x reference)
import functools

import jax
import jax.numpy as jnp
from jax import lax
from jax.experimental import pallas as pl
from jax.experimental.pallas import tpu as pltpu
from jax.experimental.pallas import tpu_sc as plsc

NC, NS = 2, 16
NW = NC * NS
BATCH = 16384
BPW = BATCH // NW
NG = BPW // 16
NVIEWS = 1000000

_MESH = plsc.VectorSubcoreMesh(
    core_axis_name="c", subcore_axis_name="s",
    num_cores=NC, num_subcores=NS,
)


@functools.partial(
    pl.kernel,
    out_type=jax.ShapeDtypeStruct((3, BATCH), jnp.float32),
    mesh=_MESH,
    scratch_types=[
        pltpu.VMEM((BPW,), jnp.int32),
        pltpu.VMEM((3, BPW), jnp.float32),
        pltpu.SemaphoreType.DMA,
    ],
    compiler_params=pltpu.CompilerParams(
        needs_layout_passes=False,
        use_tc_tiling_on_sc=False,
    ),
)
def _gather_sc(vt_hbm, idx_hbm, out_hbm, idx_v, colsT, sem):
    wid = lax.axis_index("s") * NC + lax.axis_index("c")
    pltpu.sync_copy(idx_hbm.at[pl.ds(wid * BPW, BPW)], idx_v)
    srcs = [vt_hbm.at[pl.ds(c, 1)].at[0] for c in range(3)]
    copies = []
    for t in range(NG):
        v = idx_v[pl.ds(t * 16, 16)]
        for c in range(3):
            copies.append(
                pltpu.async_copy(srcs[c].at[v],
                                 colsT.at[c, pl.ds(t * 16, 16)], sem)
            )
    for cp in copies:
        cp.wait()
    pltpu.sync_copy(colsT, out_hbm.at[:, pl.ds(wid * BPW, BPW)])


def kernel(view_correction, index):
    out = _gather_sc(view_correction.T, index)
    return out.T


# speedup vs baseline: 1.0297x; 1.0297x over previous
import functools

import jax
import jax.numpy as jnp
from jax import lax
from jax.experimental import pallas as pl
from jax.experimental.pallas import tpu as pltpu
from jax.experimental.pallas import tpu_sc as plsc

NC, NS = 2, 16
NW = NC * NS
BATCH = 16384
BPW = BATCH // NW
NG = BPW // 16
NVIEWS = 1000000

_MESH = plsc.VectorSubcoreMesh(
    core_axis_name="c", subcore_axis_name="s",
    num_cores=NC, num_subcores=NS,
)


@functools.partial(
    pl.kernel,
    out_type=jax.ShapeDtypeStruct((3, BATCH), jnp.float32),
    mesh=_MESH,
    scratch_types=[
        pltpu.VMEM((BPW,), jnp.int32),
        pltpu.VMEM((3, BPW), jnp.float32),
        pltpu.SemaphoreType.DMA,
    ],
    compiler_params=pltpu.CompilerParams(needs_layout_passes=False),
)
def _gather_sc(vt_hbm, idx_hbm, out_hbm, idx_v, colsT, sem):
    wid = lax.axis_index("s") * NC + lax.axis_index("c")
    pltpu.sync_copy(idx_hbm.at[pl.ds(wid * BPW, BPW)], idx_v)
    copies = []
    for t in range(NG):
        v = idx_v[pl.ds(t * 16, 16)]
        for c in range(3):
            copies.append(
                pltpu.async_copy(vt_hbm.at[v + (c * NVIEWS)],
                                 colsT.at[c, pl.ds(t * 16, 16)], sem)
            )
    for cp in copies:
        cp.wait()
    pltpu.sync_copy(colsT, out_hbm.at[:, pl.ds(wid * BPW, BPW)])


def kernel(view_correction, index):
    out = _gather_sc(view_correction.T.reshape(3 * NVIEWS), index)
    return out.T
